# baseline (speedup 1.0000x reference)
import functools

import jax
import jax.numpy as jnp
from jax import lax
from jax.experimental import pallas as pl
from jax.experimental.pallas import tpu as pltpu
from jax.experimental.pallas import tpu_sc as plsc

BATCH = 16384
D = 32
L = 16
NW = 32

N_U = 1000000
N_V = 100000
FULL_U = N_U // 128
FULL_V = N_V // 128
NGRP_FLAGS = (FULL_U // NW + 2) * 16


def _transpose_groups(buf4, staging, tr, flags_base, flags, idx_k):
    lane_iota = lax.iota(jnp.int32, L)
    for g in range(8):
        def extract(g=g):
            for r15 in range(16):
                col = jnp.full((L,), g * 16 + r15, jnp.int32)
                lo = plsc.load_gather(buf4, [lane_iota, col])
                hi = plsc.load_gather(buf4, [lane_iota + 16, col])
                base = g * 512 + r15 * 32
                staging[pl.ds(base, L)] = lo
                staging[pl.ds(base + L, L)] = hi
        if flags is None:
            extract()
        else:
            fv = flags[pl.ds(idx_k * 16, 16)]
            pl.when(fv[g] != 0)(extract)


def _image_loop(src3_hbm, img_hbm, bufa, bufb, staging,
                sem_a, sem_b, sem_o, *, wid, n_full, ntile, flags):
    n_own = (n_full - 1 - wid) // NW + 1

    def flag_vec(k):
        return flags[pl.ds(k * 16, 16)]

    def any_flag(k):
        if flags is None:
            return jnp.int32(1)
        fv = flag_vec(k)
        f = fv[0]
        for g in range(1, 8):
            f = f | fv[g]
        return f

    def fire_ins(k, buf, sem):
        tr = wid + k * NW
        col0 = pl.multiple_of(tr * 128, 128)
        for td in range(4):
            pltpu.async_copy(src3_hbm.at[td, :, pl.ds(col0, 128)],
                             buf.at[pl.ds(td * 8, 8), :], sem)

    def drain(sem, n, nbytes_rows):
        for _ in range(n):
            pltpu.make_async_copy(
                src3_hbm.at[0, :, pl.ds(0, 128)],
                bufa.at[pl.ds(0, 8), :], sem).wait()

    def drain_out_one():
        pltpu.make_async_copy(
            img_hbm.at[pl.ds(0, 512)],
            staging.at[pl.ds(0, 512)], sem_o).wait()

    @pl.when(jnp.logical_and(n_own > 0, any_flag(0) != 0))
    def _():
        fire_ins(0, bufa, sem_a)

    def step(k, _):
        tr = wid + k * NW
        a_k = any_flag(k)
        a_n = any_flag(k + 1)
        for parity in range(2):
            @pl.when(k % 2 == parity)
            def _(parity=parity):
                buf, sem = (bufa, sem_a) if parity == 0 else (bufb, sem_b)
                nbuf, nsem = (bufb, sem_b) if parity == 0 else (bufa, sem_a)
                if flags is None:
                    @pl.when(k >= 1)
                    def _():
                        for g in range(8):
                            drain_out_one()
                else:
                    pfv = flag_vec(jnp.maximum(k - 1, 0))
                    for g in range(8):
                        @pl.when(jnp.logical_and(k >= 1, pfv[g] != 0))
                        def _(g=g):
                            drain_out_one()
                @pl.when(jnp.logical_and(k + 1 < n_own, a_n != 0))
                def _():
                    fire_ins(k + 1, nbuf, nsem)
                @pl.when(a_k != 0)
                def _():
                    drain(sem, 4, None)
                    _transpose_groups(buf, staging, tr, None, flags, k)
                    kfv = None if flags is None else flag_vec(k)
                    for g in range(8):
                        def out(g=g):
                            base = (tr * 8 + g) * 512
                            pltpu.async_copy(
                                staging.at[pl.ds(g * 512, 512)],
                                img_hbm.at[pl.ds(base, 512)], sem_o)
                        if flags is None:
                            out()
                        else:
                            pl.when(kfv[g] != 0)(out)
        return _

    lax.fori_loop(0, n_own, step, None)

    last = jnp.maximum(n_own - 1, 0)
    lfv = None if flags is None else flags[pl.ds(last * 16, 16)]
    for g in range(8):
        if flags is None:
            @pl.when(n_own > 0)
            def _():
                drain_out_one()
        else:
            @pl.when(jnp.logical_and(n_own > 0, lfv[g] != 0))
            def _():
                drain_out_one()


def _image_body(ut3_hbm, vt3_hbm, u2_hbm, uimg_hbm, vimg_hbm,
                u2v, flags, bufa, bufb, staging,
                sem_a, sem_b, sem_o, *, nc):
    wid = lax.axis_index("s") * nc + lax.axis_index("c")
    lane_iota = lax.iota(jnp.int32, L)

    pltpu.sync_copy(u2_hbm, u2v)

    def zero(k, _):
        flags[pl.ds(k * L, L)] = jnp.zeros((L,), jnp.int32)
        return _

    lax.fori_loop(0, NGRP_FLAGS // L, zero, None)
    ones = jnp.ones((L,), jnp.int32)

    def scan(c, _):
        pos = c * L + lane_iota
        r = plsc.load_gather(u2v, [pos >> 7, pos & 127])
        tr = r >> 7
        own = (tr & (NW - 1)) == wid
        idx = (tr >> 5) * 16 + ((r >> 4) & 7)
        plsc.store_scatter(flags, [idx], ones, mask=own)
        return _

    lax.fori_loop(0, BATCH // L, scan, None)

    _image_loop(ut3_hbm, uimg_hbm, bufa, bufb, staging, sem_a, sem_b, sem_o,
                wid=wid, n_full=FULL_U, ntile=FULL_U, flags=flags)
    _image_loop(vt3_hbm, vimg_hbm, bufa, bufb, staging, sem_a, sem_b, sem_o,
                wid=wid, n_full=FULL_V, ntile=FULL_V, flags=None)


def _gather_body(u_hbm, i_hbm, j_hbm, uimg_hbm, vimg_hbm,
                 utail_hbm, vtail_hbm, out_hbm,
                 idx_u, idx_i, idx_j, urows, irows, jrows,
                 utail_v, vtail_v, s1, outv, sem, *, b_per_w, nc):
    wid = lax.axis_index("s") * nc + lax.axis_index("c")
    base = wid * b_per_w
    nrow = b_per_w // 128
    row0 = wid * nrow

    pltpu.sync_copy(u_hbm.at[pl.ds(row0, nrow)], idx_u)
    pltpu.sync_copy(i_hbm.at[pl.ds(row0, nrow)], idx_i)
    pltpu.sync_copy(j_hbm.at[pl.ds(row0, nrow)], idx_j)
    pltpu.sync_copy(utail_hbm, utail_v)
    pltpu.sync_copy(vtail_hbm, vtail_v)

    handles = []
    for c in range(nrow):
        sl = pl.ds(c * 128, 128)
        handles.append(pltpu.async_copy(
            uimg_hbm.at[idx_u.at[c]], urows.at[sl], sem))
        handles.append(pltpu.async_copy(
            vimg_hbm.at[idx_i.at[c]], irows.at[sl], sem))
        handles.append(pltpu.async_copy(
            vimg_hbm.at[idx_j.at[c]], jrows.at[sl], sem))
    for h in handles:
        h.wait()

    lane_iota = lax.iota(jnp.int32, L)

    def patch(rows_ref, idx_ref, tail_ref, limit):
        def fix(g, _):
            pos = g * L + lane_iota
            r = plsc.load_gather(idx_ref, [pos >> 7, pos & 127])
            is_tail = r >= limit
            tr = jnp.maximum(r - limit, 0)
            rows16 = pos
            for d in range(D):
                dcol = jnp.full((L,), d, jnp.int32)
                tv = plsc.load_gather(tail_ref, [tr, dcol])
                plsc.store_scatter(rows_ref, [rows16, dcol], tv, mask=is_tail)
            return _

        lax.fori_loop(0, b_per_w // L, fix, None)

    patch(urows, idx_u, utail_v, FULL_U * 128)
    patch(irows, idx_i, vtail_v, FULL_V * 128)
    patch(jrows, idx_j, vtail_v, FULL_V * 128)

    def fold(b, _):
        d0 = irows[b, pl.ds(0, L)] - jrows[b, pl.ds(0, L)]
        d1 = irows[b, pl.ds(L, L)] - jrows[b, pl.ds(L, L)]
        s1[b, :] = urows[b, pl.ds(0, L)] * d0 + urows[b, pl.ds(L, L)] * d1
        return _

    lax.fori_loop(0, b_per_w, fold, None, unroll=4)

    def reduce_grp(g, _):
        rows16 = g * L + lane_iota
        acc = plsc.load_gather(s1, [rows16, jnp.zeros((L,), jnp.int32)])
        for l in range(1, L):
            acc = acc + plsc.load_gather(s1, [rows16, jnp.full((L,), l, jnp.int32)])
        outv[pl.ds(g * L, L)] = 1.0 / (1.0 + jnp.exp(-acc))
        return _

    lax.fori_loop(0, b_per_w // L, reduce_grp, None)

    pltpu.sync_copy(outv, out_hbm.at[pl.ds(base, b_per_w)])


def kernel(u, i, j, U, V):
    try:
        info = plsc.get_sparse_core_info()
        nc, ns = info.num_cores, info.num_subcores
    except ValueError:
        nc, ns = 2, 16
    b_per_w = BATCH // (nc * ns)

    mesh = plsc.VectorSubcoreMesh(core_axis_name="c", subcore_axis_name="s")

    ut3 = U.T.reshape(4, 8, N_U)
    vt3 = V.T.reshape(4, 8, N_V)
    u2 = u.astype(jnp.int32).reshape(BATCH // 128, 128)
    i2 = i.astype(jnp.int32).reshape(BATCH // 128, 128)
    j2 = j.astype(jnp.int32).reshape(BATCH // 128, 128)

    image_k = functools.partial(
        pl.kernel,
        mesh=mesh,
        out_type=(
            jax.ShapeDtypeStruct((FULL_U * 128 * D,), jnp.float32),
            jax.ShapeDtypeStruct((FULL_V * 128 * D,), jnp.float32),
        ),
        compiler_params=pltpu.CompilerParams(
            needs_layout_passes=False, use_tc_tiling_on_sc=True
        ),
        scratch_types=[
            pltpu.VMEM((BATCH // 128, 128), jnp.int32),
            pltpu.VMEM((NGRP_FLAGS,), jnp.int32),
            pltpu.VMEM((D, 128), jnp.float32),
            pltpu.VMEM((D, 128), jnp.float32),
            pltpu.VMEM((8 * 512,), jnp.float32),
            pltpu.SemaphoreType.DMA,
            pltpu.SemaphoreType.DMA,
            pltpu.SemaphoreType.DMA,
        ],
    )(functools.partial(_image_body, nc=nc))
    uimg, vimg = image_k(ut3, vt3, u2)
    uimg2 = uimg.reshape(FULL_U * 128, D)
    vimg2 = vimg.reshape(FULL_V * 128, D)

    utail = U[FULL_U * 128:, :]
    vtail = V[FULL_V * 128:, :]

    gather_k = functools.partial(
        pl.kernel,
        mesh=mesh,
        out_type=jax.ShapeDtypeStruct((BATCH,), jnp.float32),
        compiler_params=pltpu.CompilerParams(
            needs_layout_passes=False, use_tc_tiling_on_sc=False
        ),
        scratch_types=[
            pltpu.VMEM((b_per_w // 128, 128), jnp.int32),
            pltpu.VMEM((b_per_w // 128, 128), jnp.int32),
            pltpu.VMEM((b_per_w // 128, 128), jnp.int32),
            pltpu.VMEM((b_per_w, D), jnp.float32),
            pltpu.VMEM((b_per_w, D), jnp.float32),
            pltpu.VMEM((b_per_w, D), jnp.float32),
            pltpu.VMEM((N_U - FULL_U * 128, D), jnp.float32),
            pltpu.VMEM((N_V - FULL_V * 128, D), jnp.float32),
            pltpu.VMEM((b_per_w, L), jnp.float32),
            pltpu.VMEM((b_per_w,), jnp.float32),
            pltpu.SemaphoreType.DMA,
        ],
    )(functools.partial(_gather_body, b_per_w=b_per_w, nc=nc))
    return gather_k(u2, i2, j2, uimg2, vimg2, utail, vtail)


# speedup vs baseline: 2.8632x; 2.8632x over previous
import functools

import jax
import jax.numpy as jnp
from jax import lax
from jax.experimental import pallas as pl
from jax.experimental.pallas import tpu as pltpu
from jax.experimental.pallas import tpu_sc as plsc

BATCH = 16384
D = 32
L = 16

N_U = 1000000
NTILE_U = (N_U + 127) // 128
N_FULL = N_U // 128
NBUF = 16


def _copy_body(ut3_hbm, out_hbm, bufa, bufb,
               sem_ia, sem_ib, sem_oa, sem_ob, *, nc):
    wid = lax.axis_index("s") * nc + lax.axis_index("c")
    nw = 32

    n_steps = (4 * N_FULL + nw - 1 - wid) // nw

    def src_dst(k):
        t = wid + k * nw
        td = t // N_FULL
        tr = t - td * N_FULL
        row0 = pl.multiple_of((td * NTILE_U + tr) * 8, 8)
        col0 = pl.multiple_of(tr * 128, 128)
        return td, col0, row0

    def fire_in(k, buf, sem, b):
        td, col0, _ = src_dst(k)
        pltpu.async_copy(ut3_hbm.at[td, :, pl.ds(col0, 128)],
                         buf.at[pl.ds(b * 8, 8), :], sem)

    def fire_out(k, buf, sem, b):
        _, _, row0 = src_dst(k)
        pltpu.async_copy(buf.at[pl.ds(b * 8, 8), :],
                         out_hbm.at[pl.ds(row0, 8), :], sem)

    def drain_one(sem):
        pltpu.make_async_copy(
            ut3_hbm.at[0, :, pl.ds(0, 128)], bufa.at[pl.ds(0, 8), :], sem
        ).wait()

    bank_of = (
        (bufa, sem_ia, sem_oa), (bufb, sem_ib, sem_ob))
    n_outer_val = (n_steps + NBUF - 1) // NBUF

    def outer(k2, _):
        for parity in range(2):
            @pl.when(k2 % 2 == parity)
            def _(parity=parity):
                buf, sem_i, sem_o = bank_of[parity]
                for b in range(NBUF):
                    s = (k2 - 2) * NBUF + b
                    @pl.when(jnp.logical_and(k2 >= 2, s < n_steps))
                    def _(b=b, s=s):
                        drain_one(sem_o)
                for b in range(NBUF):
                    s = k2 * NBUF + b
                    @pl.when(s < n_steps)
                    def _(b=b, s=s):
                        fire_in(s, buf, sem_i, b)
                pbuf, psem_i, psem_o = bank_of[1 - parity]
                for b in range(NBUF):
                    s = (k2 - 1) * NBUF + b
                    @pl.when(jnp.logical_and(k2 >= 1, s < n_steps))
                    def _(b=b, s=s):
                        drain_one(psem_i)
                for b in range(NBUF):
                    s = (k2 - 1) * NBUF + b
                    @pl.when(jnp.logical_and(k2 >= 1, s < n_steps))
                    def _(b=b, s=s):
                        fire_out(s, pbuf, psem_o, b)
        return _

    lax.fori_loop(0, n_outer_val + 3, outer, None)


def _gather_body(u_hbm, i_hbm, j_hbm, ulin_hbm, v_hbm, utail_hbm, out_hbm,
                 idx_u, idx_i, idx_j, widx, urows, irows, jrows, utail_v,
                 outv, sem_u, sem_v, *, b_per_w, nc):
    wid = lax.axis_index("s") * nc + lax.axis_index("c")
    base = wid * b_per_w
    nrow = b_per_w // 128
    row0 = wid * nrow

    pltpu.sync_copy(u_hbm.at[pl.ds(row0, nrow)], idx_u)
    pltpu.sync_copy(i_hbm.at[pl.ds(row0, nrow)], idx_i)
    pltpu.sync_copy(j_hbm.at[pl.ds(row0, nrow)], idx_j)
    pltpu.sync_copy(utail_hbm, utail_v)

    hv = []
    for c in range(nrow):
        sl = pl.ds(c * 128, 128)
        hv.append(pltpu.async_copy(v_hbm.at[idx_i.at[c]], irows.at[sl], sem_v))
        hv.append(pltpu.async_copy(v_hbm.at[idx_j.at[c]], jrows.at[sl], sem_v))

    lane_iota = lax.iota(jnp.int32, L)

    def widx_step(g, _):
        pos = g * L + lane_iota
        r = plsc.load_gather(idx_u, [pos >> 7, pos & 127])
        b0 = ((r >> 7) << 10) + (r & 127)
        for d in range(D):
            w = b0 + ((d // 8) * NTILE_U * 1024 + (d % 8) * 128)
            widx[d, pl.ds(g * L, L)] = w
        return _

    lax.fori_loop(0, b_per_w // L, widx_step, None)

    ngrp = b_per_w // 128
    hu = []
    for d in range(D):
        for c in range(ngrp):
            off = c * 128
            hu.append(pltpu.async_copy(
                ulin_hbm.at[widx.at[d, pl.ds(off, 128)]],
                urows.at[d, pl.ds(off, 128)], sem_u))
    for h in hv:
        h.wait()
    for h in hu:
        h.wait()

    def body(g, _):
        rows16 = g * L + lane_iota
        pos = g * L + lane_iota
        r = plsc.load_gather(idx_u, [pos >> 7, pos & 127])
        is_tail = r >= N_FULL * 128
        tr = jnp.maximum(r - N_FULL * 128, 0)
        acc = None
        for d in range(D):
            dcol = jnp.full((L,), d, jnp.int32)
            diff = (plsc.load_gather(irows, [rows16, dcol])
                    - plsc.load_gather(jrows, [rows16, dcol]))
            uv = urows[d, pl.ds(g * L, L)]
            tv = plsc.load_gather(utail_v, [tr, dcol])
            term = jnp.where(is_tail, tv, uv) * diff
            acc = term if acc is None else acc + term
        outv[pl.ds(g * L, L)] = 1.0 / (1.0 + jnp.exp(-acc))
        return _

    lax.fori_loop(0, b_per_w // L, body, None)

    pltpu.sync_copy(outv, out_hbm.at[pl.ds(base, b_per_w)])


def kernel(u, i, j, U, V):
    try:
        info = plsc.get_sparse_core_info()
        nc, ns = info.num_cores, info.num_subcores
    except ValueError:
        nc, ns = 2, 16
    nw = nc * ns
    b_per_w = BATCH // nw

    mesh = plsc.VectorSubcoreMesh(core_axis_name="c", subcore_axis_name="s")

    ut3 = U.T.reshape(4, 8, N_U)
    copy_k = functools.partial(
        pl.kernel,
        mesh=mesh,
        out_type=jax.ShapeDtypeStruct((4 * NTILE_U * 8, 128), jnp.float32),
        compiler_params=pltpu.CompilerParams(
            needs_layout_passes=False, use_tc_tiling_on_sc=True
        ),
        scratch_types=[
            pltpu.VMEM((8 * NBUF, 128), jnp.float32),
            pltpu.VMEM((8 * NBUF, 128), jnp.float32),
            pltpu.SemaphoreType.DMA,
            pltpu.SemaphoreType.DMA,
            pltpu.SemaphoreType.DMA,
            pltpu.SemaphoreType.DMA,
        ],
    )(functools.partial(_copy_body, nc=nc))
    ulin = copy_k(ut3).reshape(4 * NTILE_U * 8 * 128)

    u2 = u.astype(jnp.int32).reshape(BATCH // 128, 128)
    i2 = i.astype(jnp.int32).reshape(BATCH // 128, 128)
    j2 = j.astype(jnp.int32).reshape(BATCH // 128, 128)

    gather_k = functools.partial(
        pl.kernel,
        mesh=mesh,
        out_type=jax.ShapeDtypeStruct((BATCH,), jnp.float32),
        compiler_params=pltpu.CompilerParams(
            needs_layout_passes=False, use_tc_tiling_on_sc=False
        ),
        scratch_types=[
            pltpu.VMEM((BATCH // 128 // nw, 128), jnp.int32),
            pltpu.VMEM((BATCH // 128 // nw, 128), jnp.int32),
            pltpu.VMEM((BATCH // 128 // nw, 128), jnp.int32),
            pltpu.VMEM((D, b_per_w), jnp.int32),
            pltpu.VMEM((D, b_per_w), jnp.float32),
            pltpu.VMEM((b_per_w, D), jnp.float32),
            pltpu.VMEM((b_per_w, D), jnp.float32),
            pltpu.VMEM((N_U - N_FULL * 128, D), jnp.float32),
            pltpu.VMEM((b_per_w,), jnp.float32),
            pltpu.SemaphoreType.DMA,
            pltpu.SemaphoreType.DMA,
        ],
    )(functools.partial(_gather_body, b_per_w=b_per_w, nc=nc))
    utail = U[N_FULL * 128:, :]
    return gather_k(u2, i2, j2, ulin, V, utail)
